# baseline (speedup 1.0000x reference)
import functools

import jax
import jax.numpy as jnp
from jax import lax
from jax.experimental import pallas as pl
from jax.experimental.pallas import tpu as pltpu
from jax.experimental.pallas import tpu_sc as plsc

N_NODES = 10000
N_EDGES = 320000
D = 128

NC = 2
NS = 16
NW = NC * NS
EPW = N_EDGES // NW
K = 80
CH = EPW // K
SCH = 5
NSLAB = CH // SCH
NP = 2
HALF = 5120
TRASH = HALF
AROWS = HALF + 1
N_OUT = NP * HALF
RT = HALF // 8
DR = N_OUT // D // 2
K1 = 128
CH1 = 10240 // K1
SL1 = CH1 // SCH
R = 640
DB = R // D


def _localize(dst_s, dloc, j, lo):
    ds = []
    for v in range(K // 16):
        d = dst_s[j, pl.ds(v * 16, 16)]
        inr = (d >= lo) & (d < lo + HALF)
        dloc[pl.ds(v * 16, 16)] = jnp.where(inr, d - lo, TRASH)
        ds.append(d)
    return ds


def _sc_agg_deg_body(h_hbm, comb4, z_hbm, z1_hbm, agg_out, deg_out,
                     slab_s, dloc_v, rows_v, deg_loc, agg_sh, sem):
    cid = lax.axis_index("c")
    sid = lax.axis_index("s")
    wid = sid * NC + cid

    pltpu.sync_copy(z1_hbm, deg_loc)

    for p in range(NP):
        lo = p * HALF
        @pl.when(sid < 8)
        def _zero():
            pltpu.sync_copy(z_hbm, agg_sh.at[pl.ds(sid * RT, RT)])
        plsc.subcore_barrier()

        def chunk(g, carry):
            sl = g // SCH
            j = g % SCH
            @pl.when(j == 0)
            def _load():
                pltpu.sync_copy(comb4.at[wid, sl], slab_s)
            for v in range(K1 // 16):
                d = slab_s[j, pl.ds(K1 + v * 16, 16)]
                inr = (d >= lo) & (d < lo + HALF)
                dloc_v[pl.ds(v * 16, 16)] = jnp.where(inr, d - lo, TRASH)
                if p == 0:
                    plsc.addupdate_scatter(
                        deg_loc, [jax.lax.shift_right_logical(d, 1)],
                        jnp.where((d & 1) == 1, 65536, 1).astype(jnp.int32))
            pltpu.async_copy(h_hbm.at[slab_s.at[j, pl.ds(0, K1)]], rows_v,
                             sem).wait()
            pltpu.sync_copy(rows_v, agg_sh.at[dloc_v], add=True)
            return carry

        lax.fori_loop(0, CH1, chunk, 0)
        plsc.subcore_barrier()

        if p == 0:
            for q in range(DR):
                for k in range(D // 16):
                    rows_v[q, pl.ds(k * 16, 16)] = plsc.bitcast(
                        deg_loc[pl.ds(q * D + k * 16, 16)], jnp.float32)
            pltpu.sync_copy(rows_v.at[pl.ds(0, DR)], deg_out.at[wid])

        @pl.when(sid < 8)
        def _copyout():
            r = sid * RT
            pltpu.sync_copy(agg_sh.at[pl.ds(r, RT)],
                            agg_out.at[cid, pl.ds(lo + r, RT)])
        plsc.subcore_barrier()


def _sc_agg_body(h_hbm, src4, dst4, z_hbm, agg_out,
                 src_s, dst_s, dl0, dl1, rv0, rv1, agg_sh,
                 sg0, sg1, ss0, ss1):
    cid = lax.axis_index("c")
    sid = lax.axis_index("s")
    wid = sid * NC + cid
    rows = (rv0, rv1)
    dloc = (dl0, dl1)
    sem_g = (sg0, sg1)
    sem_s = (ss0, ss1)

    for p in range(NP):
        lo = p * HALF
        @pl.when(sid < 8)
        def _zero():
            pltpu.sync_copy(z_hbm, agg_sh.at[pl.ds(sid * RT, RT)])
        plsc.subcore_barrier()

        pltpu.sync_copy(src4.at[wid, 0], src_s)
        pltpu.sync_copy(dst4.at[wid, 0], dst_s)
        _localize(dst_s, dloc[0], 0, lo)
        g0 = pltpu.async_copy(h_hbm.at[src_s.at[0]], rows[0], sem_g[0])

        def pair(gg, carry):
            for par in (0, 1):
                g = 2 * gg + par
                nxt = g + 1
                npar = 1 - par
                def _wait_s():
                    pltpu.make_async_copy(
                        rows[npar], agg_sh.at[dloc[npar]], sem_s[npar]).wait()
                if par == 0:
                    @pl.when(gg > 0)
                    def _ws():
                        _wait_s()
                else:
                    _wait_s()
                ns = nxt // SCH
                nj = nxt % SCH
                @pl.when(nj == 0)
                def _load():
                    pltpu.make_async_copy(h_hbm.at[src_s.at[0]], rows[par],
                                          sem_g[par]).wait()
                    pltpu.sync_copy(src4.at[wid, ns], src_s)
                    pltpu.sync_copy(dst4.at[wid, ns], dst_s)
                _localize(dst_s, dloc[npar], nj, lo)
                pltpu.async_copy(h_hbm.at[src_s.at[nj]], rows[npar],
                                 sem_g[npar])
                @pl.when(nj != 0)
                def _wg():
                    pltpu.make_async_copy(h_hbm.at[src_s.at[nj]], rows[par],
                                          sem_g[par]).wait()
                pltpu.async_copy(rows[par], agg_sh.at[dloc[par]], sem_s[par],
                                 add=True)
            return carry

        lax.fori_loop(0, (CH - 1) // 2, pair, 0)
        pltpu.make_async_copy(rows[1], agg_sh.at[dloc[1]], sem_s[1]).wait()
        pltpu.make_async_copy(h_hbm.at[src_s.at[0]], rows[0], sem_g[0]).wait()
        pltpu.sync_copy(rows[0], agg_sh.at[dloc[0]], add=True)
        plsc.subcore_barrier()

        @pl.when(sid < 8)
        def _copyout():
            r = sid * RT
            pltpu.sync_copy(agg_sh.at[pl.ds(r, RT)],
                            agg_out.at[cid, pl.ds(lo + r, RT)])
        plsc.subcore_barrier()


def _make_sc(with_deg):
    mesh = plsc.VectorSubcoreMesh(core_axis_name="c", subcore_axis_name="s")
    if with_deg:
        out_type = (jax.ShapeDtypeStruct((NC, N_OUT, D), jnp.float32),
                    jax.ShapeDtypeStruct((NW, DR, D), jnp.float32))
        scratch = [
            pltpu.VMEM((SCH, 2 * K1), jnp.int32),
            pltpu.VMEM((K1,), jnp.int32),
            pltpu.VMEM((K1, D), jnp.float32),
            pltpu.VMEM((HALF,), jnp.int32),
            pltpu.VMEM_SHARED((AROWS, D), jnp.float32),
            pltpu.SemaphoreType.DMA,
        ]
        body = _sc_agg_deg_body
    else:
        out_type = (jax.ShapeDtypeStruct((NC, N_OUT, D), jnp.float32),)
        scratch = [
            pltpu.VMEM((SCH, K), jnp.int32),
            pltpu.VMEM((SCH, K), jnp.int32),
            pltpu.VMEM((K,), jnp.int32),
            pltpu.VMEM((K,), jnp.int32),
            pltpu.VMEM((K, D), jnp.float32),
            pltpu.VMEM((K, D), jnp.float32),
            pltpu.VMEM_SHARED((AROWS, D), jnp.float32),
            pltpu.SemaphoreType.DMA,
            pltpu.SemaphoreType.DMA,
            pltpu.SemaphoreType.DMA,
            pltpu.SemaphoreType.DMA,
        ]
        body = _sc_agg_body
    return pl.kernel(
        body,
        out_type=out_type,
        mesh=mesh,
        scratch_types=scratch,
        compiler_params=pltpu.CompilerParams(needs_layout_passes=False),
    )


_sc_agg_deg = _make_sc(True)
_sc_agg = _make_sc(False)


def _tc_layer_body(sig, x_ref, aggp_ref, degp_ref, e_ref, pe_ref, po_ref,
                   ws_ref, wn_ref, b_ref, o_ref):
    agg = aggp_ref[0] + aggp_ref[1]
    w = jax.lax.bitcast_convert_type(degp_ref[...], jnp.int32)
    even = jnp.sum((w & 0xFFFF).astype(jnp.float32), axis=0)
    odd = jnp.sum(jax.lax.shift_right_logical(w, 16).astype(jnp.float32),
                  axis=0)
    dcol = jnp.sum(jnp.dot(e_ref[...], even,
                           preferred_element_type=jnp.float32) * pe_ref[...]
                   + jnp.dot(e_ref[...], odd,
                             preferred_element_type=jnp.float32) * po_ref[...],
                   axis=1, keepdims=True)
    hn = agg * (1.0 / jnp.clip(dcol, 1.0, None))
    acc = (jnp.dot(x_ref[...], ws_ref[...], preferred_element_type=jnp.float32)
           + jnp.dot(hn, wn_ref[...], preferred_element_type=jnp.float32)
           + b_ref[...])
    o_ref[...] = jax.nn.sigmoid(acc) if sig else acc


def _tc_layer(x, agg_part, deg_part, emat, pemat, pomat, w_self, w_neigh, b,
              sig):
    grid = (N_OUT // R,)
    return pl.pallas_call(
        functools.partial(_tc_layer_body, sig),
        grid=grid,
        in_specs=[
            pl.BlockSpec((R, D), lambda i: (i, 0)),
            pl.BlockSpec((NC, R, D), lambda i: (0, i, 0)),
            pl.BlockSpec((NW, DR, D), lambda i: (0, 0, 0)),
            pl.BlockSpec((R, DR), lambda i: (i, 0)),
            pl.BlockSpec((R, D), lambda i: (i, 0)),
            pl.BlockSpec((R, D), lambda i: (i, 0)),
            pl.BlockSpec((D, D), lambda i: (0, 0)),
            pl.BlockSpec((D, D), lambda i: (0, 0)),
            pl.BlockSpec((D,), lambda i: (0,)),
        ],
        out_specs=pl.BlockSpec((R, D), lambda i: (i, 0)),
        out_shape=jax.ShapeDtypeStruct((N_NODES, D), jnp.float32),
    )(x, agg_part, deg_part, emat, pemat, pomat, w_self, w_neigh, b)


@jax.jit
def kernel(x, edge_index, W1_self, W1_neigh, b1, W2_self, W2_neigh, b2):
    src4 = edge_index[0].reshape(NW, NSLAB, SCH, K)
    dst4 = edge_index[1].reshape(NW, NSLAB, SCH, K)
    s2p = jnp.pad(edge_index[0].reshape(NW, EPW), ((0, 0), (0, 240)))
    d2p = jnp.pad(edge_index[1].reshape(NW, EPW), ((0, 0), (0, 240)),
                  constant_values=N_OUT - 1)
    comb4 = jnp.concatenate([s2p.reshape(NW, SL1, SCH, K1),
                             d2p.reshape(NW, SL1, SCH, K1)], axis=3)
    zeros = jnp.zeros((RT, D), jnp.float32)
    zeros1 = jnp.zeros((HALF,), jnp.int32)
    n_ids = jnp.arange(N_OUT, dtype=jnp.int32)
    emat = (n_ids[:, None] // (2 * D) == jnp.arange(DR)[None, :]
            ).astype(jnp.float32)
    half_pos = (n_ids[:, None] // 2) % D == jnp.arange(D)[None, :]
    pemat = (half_pos & (n_ids[:, None] % 2 == 0)).astype(jnp.float32)
    pomat = (half_pos & (n_ids[:, None] % 2 == 1)).astype(jnp.float32)

    agg1_part, deg_part = _sc_agg_deg(x, comb4, zeros, zeros1)
    h1 = _tc_layer(x, agg1_part, deg_part, emat, pemat, pomat,
                   W1_self, W1_neigh, b1, True)
    (agg2_part,) = _sc_agg(h1, src4, dst4, zeros)
    out = _tc_layer(h1, agg2_part, deg_part, emat, pemat, pomat,
                    W2_self, W2_neigh, b2, False)
    return out


# speedup vs baseline: 1.5868x; 1.5868x over previous
import functools

import jax
import jax.numpy as jnp
from jax import lax
from jax.experimental import pallas as pl
from jax.experimental.pallas import tpu as pltpu
from jax.experimental.pallas import tpu_sc as plsc

N_NODES = 10000
N_EDGES = 320000
D = 128

NC = 2
NS = 16
NW = NC * NS
EPW = N_EDGES // NW
K = 80
CH = EPW // K
SCH = 5
NSLAB = CH // SCH
NP = 2
HALF = 5120
TRASH = HALF
AROWS = HALF + 8
N_OUT = NP * HALF
RT = HALF // 8
DR = N_OUT // D
R = 640
DB = R // D


def _localize(dst_s, dloc, j, lo):
    ds = []
    for v in range(K // 16):
        d = dst_s[j, pl.ds(v * 16, 16)]
        inr = (d >= lo) & (d < lo + HALF)
        dloc[pl.ds(v * 16, 16)] = jnp.where(inr, d - lo, TRASH)
        ds.append(d)
    return ds


def _sc_agg_deg_body(h_hbm, src4, dst4, z_hbm, z1_hbm, agg_out, deg_out,
                     src_s, dst_s, dloc_v, rows_v, deg_loc, agg_sh, sem):
    cid = lax.axis_index("c")
    sid = lax.axis_index("s")
    wid = sid * NC + cid

    pltpu.sync_copy(z1_hbm, deg_loc)

    for p in range(NP):
        lo = p * HALF
        @pl.when(sid < 8)
        def _zero():
            pltpu.sync_copy(z_hbm, agg_sh.at[pl.ds(sid * RT, RT)])
        plsc.subcore_barrier()

        def chunk(g, carry):
            s = g // SCH
            j = g % SCH
            @pl.when(j == 0)
            def _load():
                pltpu.sync_copy(src4.at[wid, s], src_s)
                pltpu.sync_copy(dst4.at[wid, s], dst_s)
            ds = _localize(dst_s, dloc_v, j, lo)
            if p == 0:
                for d in ds:
                    plsc.addupdate_scatter(
                        deg_loc, [d], jnp.full((16,), 1.0, jnp.float32))
            pltpu.async_copy(h_hbm.at[src_s.at[j]], rows_v, sem).wait()
            pltpu.sync_copy(rows_v, agg_sh.at[dloc_v], add=True)
            return carry

        lax.fori_loop(0, CH, chunk, 0)
        plsc.subcore_barrier()

        if p == 0:
            for q in range(DR):
                for k in range(D // 16):
                    rows_v[q, pl.ds(k * 16, 16)] = (
                        deg_loc[pl.ds(q * D + k * 16, 16)])
            pltpu.sync_copy(rows_v, deg_out.at[wid])

        @pl.when(sid < 8)
        def _copyout():
            r = sid * RT
            pltpu.sync_copy(agg_sh.at[pl.ds(r, RT)],
                            agg_out.at[cid, pl.ds(lo + r, RT)])
        plsc.subcore_barrier()


def _sc_agg_body(h_hbm, src4, dst4, z_hbm, agg_out,
                 src_s, dst_s, dl0, dl1, rv0, rv1, agg_sh,
                 sg0, sg1, ss0, ss1):
    cid = lax.axis_index("c")
    sid = lax.axis_index("s")
    wid = sid * NC + cid
    rows = (rv0, rv1)
    dloc = (dl0, dl1)
    sem_g = (sg0, sg1)
    sem_s = (ss0, ss1)

    for p in range(NP):
        lo = p * HALF
        @pl.when(sid < 8)
        def _zero():
            pltpu.sync_copy(z_hbm, agg_sh.at[pl.ds(sid * RT, RT)])
        plsc.subcore_barrier()

        pltpu.sync_copy(src4.at[wid, 0], src_s)
        pltpu.sync_copy(dst4.at[wid, 0], dst_s)
        _localize(dst_s, dloc[0], 0, lo)
        g0 = pltpu.async_copy(h_hbm.at[src_s.at[0]], rows[0], sem_g[0])

        def pair(gg, carry):
            for par in (0, 1):
                g = 2 * gg + par
                nxt = g + 1
                npar = 1 - par
                def _wait_s():
                    pltpu.make_async_copy(
                        rows[npar], agg_sh.at[dloc[npar]], sem_s[npar]).wait()
                if par == 0:
                    @pl.when(gg > 0)
                    def _ws():
                        _wait_s()
                else:
                    _wait_s()
                ns = nxt // SCH
                nj = nxt % SCH
                @pl.when(nj == 0)
                def _load():
                    pltpu.make_async_copy(h_hbm.at[src_s.at[0]], rows[par],
                                          sem_g[par]).wait()
                    pltpu.sync_copy(src4.at[wid, ns], src_s)
                    pltpu.sync_copy(dst4.at[wid, ns], dst_s)
                _localize(dst_s, dloc[npar], nj, lo)
                pltpu.async_copy(h_hbm.at[src_s.at[nj]], rows[npar],
                                 sem_g[npar])
                @pl.when(nj != 0)
                def _wg():
                    pltpu.make_async_copy(h_hbm.at[src_s.at[nj]], rows[par],
                                          sem_g[par]).wait()
                pltpu.async_copy(rows[par], agg_sh.at[dloc[par]], sem_s[par],
                                 add=True)
            return carry

        lax.fori_loop(0, (CH - 1) // 2, pair, 0)
        pltpu.make_async_copy(rows[1], agg_sh.at[dloc[1]], sem_s[1]).wait()
        pltpu.make_async_copy(h_hbm.at[src_s.at[0]], rows[0], sem_g[0]).wait()
        pltpu.sync_copy(rows[0], agg_sh.at[dloc[0]], add=True)
        plsc.subcore_barrier()

        @pl.when(sid < 8)
        def _copyout():
            r = sid * RT
            pltpu.sync_copy(agg_sh.at[pl.ds(r, RT)],
                            agg_out.at[cid, pl.ds(lo + r, RT)])
        plsc.subcore_barrier()


def _make_sc(with_deg):
    mesh = plsc.VectorSubcoreMesh(core_axis_name="c", subcore_axis_name="s")
    if with_deg:
        out_type = (jax.ShapeDtypeStruct((NC, N_OUT, D), jnp.float32),
                    jax.ShapeDtypeStruct((NW, DR, D), jnp.float32))
        scratch = [
            pltpu.VMEM((SCH, K), jnp.int32),
            pltpu.VMEM((SCH, K), jnp.int32),
            pltpu.VMEM((K,), jnp.int32),
            pltpu.VMEM((K, D), jnp.float32),
            pltpu.VMEM((N_OUT,), jnp.float32),
            pltpu.VMEM_SHARED((AROWS, D), jnp.float32),
            pltpu.SemaphoreType.DMA,
        ]
        body = _sc_agg_deg_body
    else:
        out_type = (jax.ShapeDtypeStruct((NC, N_OUT, D), jnp.float32),)
        scratch = [
            pltpu.VMEM((SCH, K), jnp.int32),
            pltpu.VMEM((SCH, K), jnp.int32),
            pltpu.VMEM((K,), jnp.int32),
            pltpu.VMEM((K,), jnp.int32),
            pltpu.VMEM((K, D), jnp.float32),
            pltpu.VMEM((K, D), jnp.float32),
            pltpu.VMEM_SHARED((AROWS, D), jnp.float32),
            pltpu.SemaphoreType.DMA,
            pltpu.SemaphoreType.DMA,
            pltpu.SemaphoreType.DMA,
            pltpu.SemaphoreType.DMA,
        ]
        body = _sc_agg_body
    return pl.kernel(
        body,
        out_type=out_type,
        mesh=mesh,
        scratch_types=scratch,
        compiler_params=pltpu.CompilerParams(needs_layout_passes=False),
    )


_sc_agg_deg = _make_sc(True)
_sc_agg = _make_sc(False)


def _tc_layer_body(sig, x_ref, aggp_ref, degp_ref, e_ref, p_ref,
                   ws_ref, wn_ref, b_ref, o_ref):
    agg = aggp_ref[0] + aggp_ref[1]
    degp = jnp.sum(degp_ref[...], axis=0)
    dcol = jnp.sum(jnp.dot(e_ref[...], degp,
                           preferred_element_type=jnp.float32) * p_ref[...],
                   axis=1, keepdims=True)
    hn = agg * (1.0 / jnp.clip(dcol, 1.0, None))
    acc = (jnp.dot(x_ref[...], ws_ref[...], preferred_element_type=jnp.float32)
           + jnp.dot(hn, wn_ref[...], preferred_element_type=jnp.float32)
           + b_ref[...])
    o_ref[...] = jax.nn.sigmoid(acc) if sig else acc


def _tc_layer(x, agg_part, deg_part, emat, pmat, w_self, w_neigh, b, sig):
    grid = (N_OUT // R,)
    return pl.pallas_call(
        functools.partial(_tc_layer_body, sig),
        grid=grid,
        in_specs=[
            pl.BlockSpec((R, D), lambda i: (i, 0)),
            pl.BlockSpec((NC, R, D), lambda i: (0, i, 0)),
            pl.BlockSpec((NW, DR, D), lambda i: (0, 0, 0)),
            pl.BlockSpec((R, DR), lambda i: (i, 0)),
            pl.BlockSpec((R, D), lambda i: (0, 0)),
            pl.BlockSpec((D, D), lambda i: (0, 0)),
            pl.BlockSpec((D, D), lambda i: (0, 0)),
            pl.BlockSpec((D,), lambda i: (0,)),
        ],
        out_specs=pl.BlockSpec((R, D), lambda i: (i, 0)),
        out_shape=jax.ShapeDtypeStruct((N_NODES, D), jnp.float32),
    )(x, agg_part, deg_part, emat, pmat, w_self, w_neigh, b)


@jax.jit
def kernel(x, edge_index, W1_self, W1_neigh, b1, W2_self, W2_neigh, b2):
    src4 = edge_index[0].reshape(NW, NSLAB, SCH, K)
    dst4 = edge_index[1].reshape(NW, NSLAB, SCH, K)
    zeros = jnp.zeros((RT, D), jnp.float32)
    zeros1 = jnp.zeros((N_OUT,), jnp.float32)
    n_ids = jnp.arange(N_OUT, dtype=jnp.int32)
    emat = (n_ids[:, None] // D == jnp.arange(DR)[None, :]).astype(jnp.float32)
    pmat = (n_ids[:R, None] % D == jnp.arange(D)[None, :]).astype(jnp.float32)

    agg1_part, deg_part = _sc_agg_deg(x, src4, dst4, zeros, zeros1)
    h1 = _tc_layer(x, agg1_part, deg_part, emat, pmat, W1_self, W1_neigh, b1,
                   True)
    (agg2_part,) = _sc_agg(h1, src4, dst4, zeros)
    out = _tc_layer(h1, agg2_part, deg_part, emat, pmat, W2_self, W2_neigh, b2,
                    False)
    return out


# speedup vs baseline: 1.7487x; 1.1021x over previous
import functools

import jax
import jax.numpy as jnp
from jax import lax
from jax.experimental import pallas as pl
from jax.experimental.pallas import tpu as pltpu
from jax.experimental.pallas import tpu_sc as plsc

N_NODES = 10000
N_EDGES = 320000
D = 128

NC = 2
NS = 16
NW = NC * NS
EPW = N_EDGES // NW
K = 80
CH = EPW // K
SCH = 5
NSLAB = CH // SCH
NP = 2
HALF = 5120
TRASH = HALF
AROWS = HALF + 8
N_OUT = NP * HALF
RT = HALF // 8
DR = N_OUT // D
R = 640
DB = R // D


def _localize(dst_s, dloc, j, lo):
    ds = []
    for v in range(K // 16):
        d = dst_s[j, pl.ds(v * 16, 16)]
        inr = (d >= lo) & (d < lo + HALF)
        dloc[pl.ds(v * 16, 16)] = jnp.where(inr, d - lo, TRASH)
        ds.append(d)
    return ds


SUBS = (48, 32)


def _sc_agg_deg_body(h_hbm, src4, dst4, z_hbm, z1_hbm, agg_out, deg_out,
                     src_s, dst_s, dlA, dlB, rvA, rvB, deg_loc, agg_sh,
                     sgA, sgB, ssA, ssB):
    cid = lax.axis_index("c")
    sid = lax.axis_index("s")
    wid = sid * NC + cid
    rows = (rvA, rvB)
    dloc = (dlA, dlB)
    sem_g = (sgA, sgB)
    sem_s = (ssA, ssB)
    OFF = (0, SUBS[0])

    pltpu.sync_copy(z1_hbm, deg_loc)

    def _loc(j, h, lo, p):
        nv = SUBS[h] // 16
        for v in range(nv):
            d = dst_s[j, pl.ds(OFF[h] + v * 16, 16)]
            inr = (d >= lo) & (d < lo + HALF)
            dloc[h][pl.ds(v * 16, 16)] = jnp.where(inr, d - lo, TRASH)
            if p == 0:
                plsc.addupdate_scatter(
                    deg_loc, [d], jnp.full((16,), 1.0, jnp.float32))

    def _gather(j, h):
        pltpu.async_copy(h_hbm.at[src_s.at[j, pl.ds(OFF[h], SUBS[h])]],
                         rows[h], sem_g[h])

    def _wait_g(h):
        pltpu.make_async_copy(h_hbm.at[src_s.at[0, pl.ds(OFF[h], SUBS[h])]],
                              rows[h], sem_g[h]).wait()

    def _scatter(h):
        pltpu.async_copy(rows[h], agg_sh.at[dloc[h]], sem_s[h], add=True)

    def _wait_s(h):
        pltpu.make_async_copy(rows[h], agg_sh.at[dloc[h]], sem_s[h]).wait()

    for p in range(NP):
        lo = p * HALF
        @pl.when(sid < 8)
        def _zero():
            pltpu.sync_copy(z_hbm, agg_sh.at[pl.ds(sid * RT, RT)])
        plsc.subcore_barrier()

        pltpu.sync_copy(src4.at[wid, 0], src_s)
        pltpu.sync_copy(dst4.at[wid, 0], dst_s)
        _loc(0, 0, lo, p)
        _gather(0, 0)

        def chunk(c, carry):
            j = c % SCH
            @pl.when(c > 0)
            def _wsb():
                _wait_s(1)
            _loc(j, 1, lo, p)
            _gather(j, 1)
            _wait_g(0)
            _scatter(0)
            _wait_s(0)
            nc_ = c + 1
            nj = nc_ % SCH
            @pl.when((nj == 0) & (nc_ < CH))
            def _load():
                _wait_g(1)
                pltpu.sync_copy(src4.at[wid, nc_ // SCH], src_s)
                pltpu.sync_copy(dst4.at[wid, nc_ // SCH], dst_s)
            @pl.when(nc_ < CH)
            def _nextA():
                _loc(nj, 0, lo, p)
                _gather(nj, 0)
            @pl.when(nj != 0)
            def _wgb():
                _wait_g(1)
            _scatter(1)
            return carry

        lax.fori_loop(0, CH, chunk, 0)
        _wait_s(1)
        plsc.subcore_barrier()

        if p == 0:
            for q in range(DR):
                for k in range(D // 16):
                    buf = rvA if q < SUBS[0] else rvB
                    qq = q if q < SUBS[0] else q - SUBS[0]
                    buf[qq, pl.ds(k * 16, 16)] = (
                        deg_loc[pl.ds(q * D + k * 16, 16)])
            pltpu.sync_copy(rvA, deg_out.at[wid, pl.ds(0, SUBS[0])])
            pltpu.sync_copy(rvB, deg_out.at[wid, pl.ds(SUBS[0], SUBS[1])])

        @pl.when(sid < 8)
        def _copyout():
            r = sid * RT
            pltpu.sync_copy(agg_sh.at[pl.ds(r, RT)],
                            agg_out.at[cid, pl.ds(lo + r, RT)])
        plsc.subcore_barrier()


def _sc_agg_body(h_hbm, src4, dst4, z_hbm, agg_out,
                 src_s, dst_s, dl0, dl1, rv0, rv1, agg_sh,
                 sg0, sg1, ss0, ss1):
    cid = lax.axis_index("c")
    sid = lax.axis_index("s")
    wid = sid * NC + cid
    rows = (rv0, rv1)
    dloc = (dl0, dl1)
    sem_g = (sg0, sg1)
    sem_s = (ss0, ss1)

    for p in range(NP):
        lo = p * HALF
        @pl.when(sid < 8)
        def _zero():
            pltpu.sync_copy(z_hbm, agg_sh.at[pl.ds(sid * RT, RT)])
        plsc.subcore_barrier()

        pltpu.sync_copy(src4.at[wid, 0], src_s)
        pltpu.sync_copy(dst4.at[wid, 0], dst_s)
        _localize(dst_s, dloc[0], 0, lo)
        g0 = pltpu.async_copy(h_hbm.at[src_s.at[0]], rows[0], sem_g[0])

        def pair(gg, carry):
            for par in (0, 1):
                g = 2 * gg + par
                nxt = g + 1
                npar = 1 - par
                def _wait_s():
                    pltpu.make_async_copy(
                        rows[npar], agg_sh.at[dloc[npar]], sem_s[npar]).wait()
                if par == 0:
                    @pl.when(gg > 0)
                    def _ws():
                        _wait_s()
                else:
                    _wait_s()
                ns = nxt // SCH
                nj = nxt % SCH
                @pl.when(nj == 0)
                def _load():
                    pltpu.make_async_copy(h_hbm.at[src_s.at[0]], rows[par],
                                          sem_g[par]).wait()
                    pltpu.sync_copy(src4.at[wid, ns], src_s)
                    pltpu.sync_copy(dst4.at[wid, ns], dst_s)
                _localize(dst_s, dloc[npar], nj, lo)
                pltpu.async_copy(h_hbm.at[src_s.at[nj]], rows[npar],
                                 sem_g[npar])
                @pl.when(nj != 0)
                def _wg():
                    pltpu.make_async_copy(h_hbm.at[src_s.at[nj]], rows[par],
                                          sem_g[par]).wait()
                pltpu.async_copy(rows[par], agg_sh.at[dloc[par]], sem_s[par],
                                 add=True)
            return carry

        lax.fori_loop(0, (CH - 1) // 2, pair, 0)
        pltpu.make_async_copy(rows[1], agg_sh.at[dloc[1]], sem_s[1]).wait()
        pltpu.make_async_copy(h_hbm.at[src_s.at[0]], rows[0], sem_g[0]).wait()
        pltpu.sync_copy(rows[0], agg_sh.at[dloc[0]], add=True)
        plsc.subcore_barrier()

        @pl.when(sid < 8)
        def _copyout():
            r = sid * RT
            pltpu.sync_copy(agg_sh.at[pl.ds(r, RT)],
                            agg_out.at[cid, pl.ds(lo + r, RT)])
        plsc.subcore_barrier()


def _make_sc(with_deg):
    mesh = plsc.VectorSubcoreMesh(core_axis_name="c", subcore_axis_name="s")
    if with_deg:
        out_type = (jax.ShapeDtypeStruct((NC, N_OUT, D), jnp.float32),
                    jax.ShapeDtypeStruct((NW, DR, D), jnp.float32))
        scratch = [
            pltpu.VMEM((SCH, K), jnp.int32),
            pltpu.VMEM((SCH, K), jnp.int32),
            pltpu.VMEM((48,), jnp.int32),
            pltpu.VMEM((32,), jnp.int32),
            pltpu.VMEM((48, D), jnp.float32),
            pltpu.VMEM((32, D), jnp.float32),
            pltpu.VMEM((N_OUT,), jnp.float32),
            pltpu.VMEM_SHARED((AROWS, D), jnp.float32),
            pltpu.SemaphoreType.DMA,
            pltpu.SemaphoreType.DMA,
            pltpu.SemaphoreType.DMA,
            pltpu.SemaphoreType.DMA,
        ]
        body = _sc_agg_deg_body
    else:
        out_type = (jax.ShapeDtypeStruct((NC, N_OUT, D), jnp.float32),)
        scratch = [
            pltpu.VMEM((SCH, K), jnp.int32),
            pltpu.VMEM((SCH, K), jnp.int32),
            pltpu.VMEM((K,), jnp.int32),
            pltpu.VMEM((K,), jnp.int32),
            pltpu.VMEM((K, D), jnp.float32),
            pltpu.VMEM((K, D), jnp.float32),
            pltpu.VMEM_SHARED((AROWS, D), jnp.float32),
            pltpu.SemaphoreType.DMA,
            pltpu.SemaphoreType.DMA,
            pltpu.SemaphoreType.DMA,
            pltpu.SemaphoreType.DMA,
        ]
        body = _sc_agg_body
    return pl.kernel(
        body,
        out_type=out_type,
        mesh=mesh,
        scratch_types=scratch,
        compiler_params=pltpu.CompilerParams(needs_layout_passes=False),
    )


_sc_agg_deg = _make_sc(True)
_sc_agg = _make_sc(False)


def _tc_layer_body(sig, x_ref, aggp_ref, degp_ref, e_ref, p_ref,
                   ws_ref, wn_ref, b_ref, o_ref):
    agg = aggp_ref[0] + aggp_ref[1]
    degp = jnp.sum(degp_ref[...], axis=0)
    dcol = jnp.sum(jnp.dot(e_ref[...], degp,
                           preferred_element_type=jnp.float32) * p_ref[...],
                   axis=1, keepdims=True)
    hn = agg * (1.0 / jnp.clip(dcol, 1.0, None))
    acc = (jnp.dot(x_ref[...], ws_ref[...], preferred_element_type=jnp.float32)
           + jnp.dot(hn, wn_ref[...], preferred_element_type=jnp.float32)
           + b_ref[...])
    o_ref[...] = jax.nn.sigmoid(acc) if sig else acc


def _tc_layer(x, agg_part, deg_part, emat, pmat, w_self, w_neigh, b, sig):
    grid = (N_OUT // R,)
    return pl.pallas_call(
        functools.partial(_tc_layer_body, sig),
        grid=grid,
        in_specs=[
            pl.BlockSpec((R, D), lambda i: (i, 0)),
            pl.BlockSpec((NC, R, D), lambda i: (0, i, 0)),
            pl.BlockSpec((NW, DR, D), lambda i: (0, 0, 0)),
            pl.BlockSpec((R, DR), lambda i: (i, 0)),
            pl.BlockSpec((R, D), lambda i: (0, 0)),
            pl.BlockSpec((D, D), lambda i: (0, 0)),
            pl.BlockSpec((D, D), lambda i: (0, 0)),
            pl.BlockSpec((D,), lambda i: (0,)),
        ],
        out_specs=pl.BlockSpec((R, D), lambda i: (i, 0)),
        out_shape=jax.ShapeDtypeStruct((N_NODES, D), jnp.float32),
    )(x, agg_part, deg_part, emat, pmat, w_self, w_neigh, b)


@jax.jit
def kernel(x, edge_index, W1_self, W1_neigh, b1, W2_self, W2_neigh, b2):
    src4 = edge_index[0].reshape(NW, NSLAB, SCH, K)
    dst4 = edge_index[1].reshape(NW, NSLAB, SCH, K)
    zeros = jnp.zeros((RT, D), jnp.float32)
    zeros1 = jnp.zeros((N_OUT,), jnp.float32)
    n_ids = jnp.arange(N_OUT, dtype=jnp.int32)
    emat = (n_ids[:, None] // D == jnp.arange(DR)[None, :]).astype(jnp.float32)
    pmat = (n_ids[:R, None] % D == jnp.arange(D)[None, :]).astype(jnp.float32)

    agg1_part, deg_part = _sc_agg_deg(x, src4, dst4, zeros, zeros1)
    h1 = _tc_layer(x, agg1_part, deg_part, emat, pmat, W1_self, W1_neigh, b1,
                   True)
    (agg2_part,) = _sc_agg(h1, src4, dst4, zeros)
    out = _tc_layer(h1, agg2_part, deg_part, emat, pmat, W2_self, W2_neigh, b2,
                    False)
    return out
